# baseline (speedup 1.0000x reference)
import jax
import jax.numpy as jnp
from jax.experimental import pallas as pl
from jax.experimental.pallas import tpu as pltpu

_NCLS = 100000
_B = 1024
_RB = 8
_ALIGN = (_NCLS // 128) * 128
_LN2 = 0.6931471805599453


def _row_body(x_ref, tcol_ref, lse_ref, tv_ref):
    xa = x_ref[:, :_ALIGN]
    xt = x_ref[:, _ALIGN:_NCLS]
    m = jnp.maximum(
        jnp.max(xa, axis=1, keepdims=True), jnp.max(xt, axis=1, keepdims=True)
    )
    s = jnp.sum(jnp.exp(xa - m), axis=1, keepdims=True) + jnp.sum(
        jnp.exp(xt - m), axis=1, keepdims=True
    )
    lse_ref[...] = m + jnp.log(s)

    tcol = tcol_ref[...]
    cols_a = jax.lax.broadcasted_iota(jnp.int32, (1, _ALIGN), 1)
    cols_t = _ALIGN + jax.lax.broadcasted_iota(jnp.int32, (1, _NCLS - _ALIGN), 1)
    tv_ref[...] = jnp.sum(
        jnp.where(cols_a == tcol, xa, 0.0), axis=1, keepdims=True
    ) + jnp.sum(jnp.where(cols_t == tcol, xt, 0.0), axis=1, keepdims=True)


def _combine_body(lse_ref, tv_ref, tcol_ref, trow_ref, out_ref):
    nll = lse_ref[...] - tv_ref[...]
    cnt = jnp.sum(
        (tcol_ref[...] == trow_ref[...]).astype(jnp.float32), axis=1, keepdims=True
    )
    u = 1.0 / (1.0 - jnp.exp(cnt * (-_LN2)))
    num = jnp.sum(u * nll, axis=(0, 1), keepdims=True)
    den = jnp.sum(u, axis=(0, 1), keepdims=True)
    out_ref[...] = num / den


def kernel(inputs, targets):
    x = inputs.reshape(_B, _NCLS)
    t = targets.reshape(-1).astype(jnp.int32)
    tcol = t.reshape(_B, 1)
    trow = t.reshape(1, _B)

    lse, tv = pl.pallas_call(
        _row_body,
        grid=(_B // _RB,),
        in_specs=[
            pl.BlockSpec((_RB, _NCLS), lambda i: (i, 0)),
            pl.BlockSpec((_RB, 1), lambda i: (i, 0)),
        ],
        out_specs=[
            pl.BlockSpec((_RB, 1), lambda i: (i, 0)),
            pl.BlockSpec((_RB, 1), lambda i: (i, 0)),
        ],
        out_shape=[
            jax.ShapeDtypeStruct((_B, 1), jnp.float32),
            jax.ShapeDtypeStruct((_B, 1), jnp.float32),
        ],
        compiler_params=pltpu.CompilerParams(
            dimension_semantics=("arbitrary",),
        ),
    )(x, tcol)

    out = pl.pallas_call(
        _combine_body,
        out_shape=jax.ShapeDtypeStruct((1, 1), jnp.float32),
    )(lse, tv, tcol, trow)
    return out[0, 0]


# speedup vs baseline: 1.1892x; 1.1892x over previous
import jax
import jax.numpy as jnp
from jax.experimental import pallas as pl
from jax.experimental.pallas import tpu as pltpu

_NCLS = 100000
_B = 1024
_CB = 4096
_NBLK = (_NCLS + _CB - 1) // _CB
_NQ = 4
_QB = _B // _NQ
_LN2 = 0.6931471805599453


def _body(x0, x1, x2, x3, tcol_ref, trow_ref, out_ref, m_ref, s_ref, tv_ref):
    j = pl.program_id(0)

    @pl.when(j == 0)
    def _init():
        m_ref[...] = jnp.full((_B, 1), -jnp.inf, jnp.float32)
        s_ref[...] = jnp.zeros((_B, 1), jnp.float32)
        tv_ref[...] = jnp.zeros((_B, 1), jnp.float32)

    def _update(x_refs, mask_tail):
        col_ids = j * _CB + jax.lax.broadcasted_iota(jnp.int32, (1, _CB), 1)
        for q, xr in enumerate(x_refs):
            sl = slice(q * _QB, (q + 1) * _QB)
            x = xr[...]
            xm = jnp.where(col_ids < _NCLS, x, -jnp.inf) if mask_tail else x
            bm = jnp.max(xm, axis=1, keepdims=True)
            m_old = m_ref[sl, :]
            m_new = jnp.maximum(m_old, bm)
            s_ref[sl, :] = s_ref[sl, :] * jnp.exp(m_old - m_new) + jnp.sum(
                jnp.exp(xm - m_new), axis=1, keepdims=True
            )
            m_ref[sl, :] = m_new
            hit = col_ids == tcol_ref[sl, :]
            tv_ref[sl, :] += jnp.sum(jnp.where(hit, x, 0.0), axis=1, keepdims=True)

    @pl.when(j < _NBLK - 1)
    def _main():
        _update((x0, x1, x2, x3), False)

    @pl.when(j == _NBLK - 1)
    def _tail():
        _update((x0, x1, x2, x3), True)

    @pl.when(j == _NBLK - 1)
    def _fin():
        lse = m_ref[...] + jnp.log(s_ref[...])
        nll = lse - tv_ref[...]
        cnt = jnp.sum(
            (tcol_ref[...] == trow_ref[...]).astype(jnp.float32),
            axis=1,
            keepdims=True,
        )
        u = 1.0 / (1.0 - jnp.exp(cnt * (-_LN2)))
        num = jnp.sum(u * nll, axis=(0, 1), keepdims=True)
        den = jnp.sum(u, axis=(0, 1), keepdims=True)
        out_ref[...] = num / den


def kernel(inputs, targets):
    x = inputs.reshape(_B, _NCLS)
    t = targets.reshape(-1).astype(jnp.int32)
    tcol = t.reshape(_B, 1)
    trow = t.reshape(1, _B)

    x_specs = [
        pl.BlockSpec((_QB, _CB), lambda j, q=q: (q, j)) for q in range(_NQ)
    ]
    out = pl.pallas_call(
        _body,
        grid=(_NBLK,),
        in_specs=x_specs
        + [
            pl.BlockSpec((_B, 1), lambda j: (0, 0)),
            pl.BlockSpec((1, _B), lambda j: (0, 0)),
        ],
        out_specs=pl.BlockSpec((1, 1), lambda j: (0, 0)),
        out_shape=jax.ShapeDtypeStruct((1, 1), jnp.float32),
        scratch_shapes=[
            pltpu.VMEM((_B, 1), jnp.float32),
            pltpu.VMEM((_B, 1), jnp.float32),
            pltpu.VMEM((_B, 1), jnp.float32),
        ],
        compiler_params=pltpu.CompilerParams(
            dimension_semantics=("arbitrary",),
        ),
    )(x, x, x, x, tcol, trow)
    return out[0, 0]
